# baseline (speedup 1.0000x reference)
import functools
import jax
import jax.numpy as jnp
from jax import lax
from jax.experimental import pallas as pl
from jax.experimental.pallas import tpu as pltpu
from jax.experimental.pallas import tpu_sc as plsc

N = 10000
E = 320000
D = 128
NPAD = 10240
HROWS = 80
BR = 2048
GRID_N = NPAD // BR
NC = 2
NS = 16
NW = NC * NS
CH = 64
NCHUNKS = E // CH
CHUNKS_BASE = NCHUNKS // NW
CHUNKS_REM = NCHUNKS % NW
EPT = E // NW
ROWS_PER_TILE = NPAD // NS

@functools.lru_cache(maxsize=None)
def _sc_mesh():
    return plsc.VectorSubcoreMesh(core_axis_name="c", subcore_axis_name="s",
                                  num_cores=NC, num_subcores=NS)


def _deg_body(src_ref, dst_ref, deg_ref, hist, idxbuf, rowA, rowB, accum, sem):
    c = lax.axis_index("c")
    s = lax.axis_index("s")
    wid = s * NC + c

    def zero_hist():
        def zrow(i, _):
            for j in range(8):
                hist[i, pl.ds(j * 16, 16)] = jnp.zeros((16,), jnp.float32)
            return _
        lax.fori_loop(0, HROWS, zrow, None)

    zero_hist()
    for j in range(5):
        base = lax.iota(jnp.int32, 16) + j * 16
        rowA[pl.ds(j * 16, 16)] = base
        rowB[pl.ds(j * 16, 16)] = base + HROWS
    @pl.when(s < 10)
    def _():
        pltpu.sync_copy(hist.at[pl.ds(0, 16)], accum.at[pl.ds(s * 16, 16)])
    plsc.subcore_barrier()

    def histogram(idx_hbm_ref, row_ref):
        pltpu.async_copy(idx_hbm_ref, idxbuf, sem).wait()

        def step(g, _):
            v = idxbuf[pl.ds(g * 16, 16)]
            hi = lax.shift_right_logical(v, 7)
            lo = lax.bitwise_and(v, 127)
            cnt, last = plsc.scan_count(v)
            val = cnt.astype(jnp.float32)
            plsc.addupdate_scatter(hist, [hi, lo], val, mask=last)
            return _
        lax.fori_loop(0, EPT // 16, step, None)
        pltpu.sync_copy(hist, accum.at[row_ref], add=True)

    histogram(src_ref.at[pl.ds(wid * EPT, EPT)], rowA)
    zero_hist()
    histogram(dst_ref.at[pl.ds(wid * EPT, EPT)], rowB)
    plsc.subcore_barrier()

    @pl.when(s < 10)
    def _():
        pltpu.sync_copy(accum.at[pl.ds(s * 16, 16)],
                        deg_ref.at[c, pl.ds(s * 16, 16)])


@functools.lru_cache(maxsize=None)
def _deg_kernel():
  return pl.kernel(
    _deg_body,
    out_type=jax.ShapeDtypeStruct((NC, 2 * HROWS, D), jnp.float32),
    mesh=_sc_mesh(),
    scratch_types=[
        pltpu.VMEM((HROWS, D), jnp.float32),
        pltpu.VMEM((EPT,), jnp.int32),
        pltpu.VMEM((HROWS,), jnp.int32),
        pltpu.VMEM((HROWS,), jnp.int32),
        pltpu.VMEM_SHARED((2 * HROWS, D), jnp.float32),
        pltpu.SemaphoreType.DMA,
    ],
    compiler_params=pltpu.CompilerParams(needs_layout_passes=False),
  )


def _b1_body(x_ref, ws_ref, bs_ref, wd_ref, bd_ref, deg_ref,
             g_ref, e2_ref, nr_ref):
    x = x_ref[...]
    g_ref[:, :D] = jnp.dot(x, ws_ref[...], preferred_element_type=jnp.float32,
                           precision=lax.Precision.HIGHEST) + bs_ref[...][None, :]
    e2_ref[...] = jnp.dot(x, wd_ref[...], preferred_element_type=jnp.float32,
                          precision=lax.Precision.HIGHEST) + bd_ref[...][None, :]
    deg = deg_ref[...]
    out_deg = deg[0, 0] + deg[1, 0]
    in_deg = deg[0, 1] + deg[1, 1]
    norm_l = lax.rsqrt(jnp.maximum(out_deg, 1.0))
    nr_ref[...] = lax.rsqrt(jnp.maximum(in_deg, 1.0))
    g_ref[:, D:] = x * norm_l


def _run_b1(x_pad, W_src, b_src, W_dst, b_dst, deg4):
    f32 = jnp.float32
    return pl.pallas_call(
        _b1_body,
        grid=(GRID_N,),
        in_specs=[
            pl.BlockSpec((BR, D), lambda i: (i, 0)),
            pl.BlockSpec((D, D), lambda i: (0, 0)),
            pl.BlockSpec((D,), lambda i: (0,)),
            pl.BlockSpec((D, D), lambda i: (0, 0)),
            pl.BlockSpec((D,), lambda i: (0,)),
            pl.BlockSpec((NC, 2, BR, 1), lambda i: (0, 0, i, 0)),
        ],
        out_specs=[
            pl.BlockSpec((BR, 2 * D), lambda i: (i, 0)),
            pl.BlockSpec((BR, D), lambda i: (i, 0)),
            pl.BlockSpec((BR, 1), lambda i: (i, 0)),
        ],
        out_shape=[
            jax.ShapeDtypeStruct((NPAD, 2 * D), f32),
            jax.ShapeDtypeStruct((NPAD, D), f32),
            jax.ShapeDtypeStruct((NPAD, 1), f32),
        ],
    )(x_pad, W_src, b_src, W_dst, b_dst, deg4)


def _b2_body(ef_ref, we_ref, be_ref, et_ref):
    et_ref[...] = jnp.dot(ef_ref[...], we_ref[...],
                          preferred_element_type=jnp.float32,
                          precision=lax.Precision.HIGHEST) + be_ref[...][None, :]


def _run_b2(edge_feats, W_edge, b_edge):
    BE = 8000
    return pl.pallas_call(
        _b2_body,
        grid=(E // BE,),
        in_specs=[
            pl.BlockSpec((BE, D), lambda i: (i, 0)),
            pl.BlockSpec((D, D), lambda i: (0, 0)),
            pl.BlockSpec((D,), lambda i: (0,)),
        ],
        out_specs=pl.BlockSpec((BE, D), lambda i: (i, 0)),
        out_shape=jax.ShapeDtypeStruct((E, D), jnp.float32),
    )(edge_feats, W_edge, b_edge)


def _edge_body(src_ref, dst_ref, g_ref, e2_ref, et_ref,
               m_ref, part_ref,
               bAF, bB, bET, sidx, didx, bufM, accum, sem):
    c = lax.axis_index("c")
    s = lax.axis_index("s")
    wid = s * NC + c

    def zrow(i, _):
        for j in range(8):
            bufM[i, pl.ds(j * 16, 16)] = jnp.zeros((16,), jnp.float32)
        return _
    lax.fori_loop(0, CH, zrow, None)
    r0 = s * ROWS_PER_TILE
    nfull = ROWS_PER_TILE // CH
    remr = ROWS_PER_TILE % CH
    for r in range(nfull):
        pltpu.sync_copy(bufM, accum.at[pl.ds(r0 + r * CH, CH)])
    if remr:
        pltpu.sync_copy(bufM.at[pl.ds(0, remr)],
                        accum.at[pl.ds(r0 + nfull * CH, remr)])
    plsc.subcore_barrier()

    nchunks_w = CHUNKS_BASE + (wid < CHUNKS_REM).astype(jnp.int32)

    def chunk(j, _):
        eoff = (wid + NW * j) * CH
        pltpu.sync_copy(src_ref.at[pl.ds(eoff, CH)], sidx)
        pltpu.sync_copy(dst_ref.at[pl.ds(eoff, CH)], didx)
        d1 = pltpu.async_copy(g_ref.at[sidx], bAF, sem)
        d2 = pltpu.async_copy(e2_ref.at[didx], bB, sem)
        d3 = pltpu.async_copy(et_ref.at[pl.ds(eoff, CH)], bET, sem)
        d1.wait(); d2.wait(); d3.wait()

        def row(i, _):
            for j8 in range(8):
                sl = pl.ds(j8 * 16, 16)
                m = bAF[i, sl] + bB[i, sl] + bET[i, sl]
                sig = 1.0 / (1.0 + jnp.exp(-m))
                bET[i, sl] = m
                bufM[i, sl] = bAF[i, pl.ds(D + j8 * 16, 16)] * sig
            return _
        lax.fori_loop(0, CH, row, None)

        pltpu.sync_copy(bET, m_ref.at[pl.ds(eoff, CH)])
        pltpu.sync_copy(bufM, accum.at[didx], add=True)
        return _
    lax.fori_loop(0, nchunks_w, chunk, None)

    plsc.subcore_barrier()
    pltpu.sync_copy(accum.at[pl.ds(r0, ROWS_PER_TILE)],
                    part_ref.at[c, pl.ds(r0, ROWS_PER_TILE)])


@functools.lru_cache(maxsize=None)
def _edge_kernel():
  f32 = jnp.float32
  return pl.kernel(
    _edge_body,
    out_type=(
        jax.ShapeDtypeStruct((E, D), f32),
        jax.ShapeDtypeStruct((NC, NPAD, D), f32),
    ),
    mesh=_sc_mesh(),
    scratch_types=[
        pltpu.VMEM((CH, 2 * D), f32),
        pltpu.VMEM((CH, D), f32),
        pltpu.VMEM((CH, D), f32),
        pltpu.VMEM((CH,), jnp.int32),
        pltpu.VMEM((CH,), jnp.int32),
        pltpu.VMEM((CH, D), f32),
        pltpu.VMEM_SHARED((NPAD, D), f32),
        pltpu.SemaphoreType.DMA,
    ],
  )


def _d_body(part_ref, w_ref, b_ref, nr_ref, x_ref, out_ref):
    p = part_ref[...]
    sacc = p[0] + p[1]
    r = jnp.dot(sacc, w_ref[...], preferred_element_type=jnp.float32,
                precision=lax.Precision.HIGHEST)
    out_ref[...] = r * nr_ref[...] + b_ref[...][None, :] + x_ref[...]


def _run_d(partials, weight, bias, nr, x_pad):
    return pl.pallas_call(
        _d_body,
        grid=(GRID_N,),
        in_specs=[
            pl.BlockSpec((NC, BR, D), lambda i: (0, i, 0)),
            pl.BlockSpec((D, D), lambda i: (0, 0)),
            pl.BlockSpec((D,), lambda i: (0,)),
            pl.BlockSpec((BR, 1), lambda i: (i, 0)),
            pl.BlockSpec((BR, D), lambda i: (i, 0)),
        ],
        out_specs=pl.BlockSpec((BR, D), lambda i: (i, 0)),
        out_shape=jax.ShapeDtypeStruct((NPAD, D), jnp.float32),
    )(partials, weight, bias, nr, x_pad)


@jax.jit
def kernel(node_feats, edge_index, edge_feats, weight, bias,
           W_src, b_src, W_dst, b_dst, W_edge, b_edge):
    src = edge_index[0].astype(jnp.int32)
    dst = edge_index[1].astype(jnp.int32)
    x_pad = jnp.pad(node_feats, ((0, NPAD - N), (0, 0)))

    deg = _deg_kernel()(src, dst)
    deg4 = deg.reshape(NC, 2, NPAD, 1)
    g, e2, nr = _run_b1(x_pad, W_src, b_src, W_dst, b_dst, deg4)
    et = _run_b2(edge_feats, W_edge, b_edge)
    m, partials = _edge_kernel()(src, dst, g, e2, et)
    rst_pad = _run_d(partials, weight, bias, nr, x_pad)
    return (rst_pad[:N], m)


# speedup vs baseline: 2.5775x; 2.5775x over previous
import functools
import jax
import jax.numpy as jnp
from jax import lax
from jax.experimental import pallas as pl
from jax.experimental.pallas import tpu as pltpu
from jax.experimental.pallas import tpu_sc as plsc

N = 10000
E = 320000
D = 128
NPAD = 10240
HROWS = 80
BR = 2048
GRID_N = NPAD // BR
NC = 2
NS = 16
NW = NC * NS
CH = 64
NCHUNKS = E // CH
CHUNKS_BASE = NCHUNKS // NW
CHUNKS_REM = NCHUNKS % NW
EPT = E // NW
ROWS_PER_TILE = NPAD // NS

@functools.lru_cache(maxsize=None)
def _sc_mesh():
    return plsc.VectorSubcoreMesh(core_axis_name="c", subcore_axis_name="s",
                                  num_cores=NC, num_subcores=NS)


def _deg_body(src_ref, dst_ref, deg_ref, hist, idxbuf, rowA, rowB, accum, sem):
    c = lax.axis_index("c")
    s = lax.axis_index("s")
    wid = s * NC + c

    def zero_hist():
        def zrow(i, _):
            for j in range(8):
                hist[i, pl.ds(j * 16, 16)] = jnp.zeros((16,), jnp.float32)
            return _
        lax.fori_loop(0, HROWS, zrow, None)

    zero_hist()
    for j in range(5):
        base = lax.iota(jnp.int32, 16) + j * 16
        rowA[pl.ds(j * 16, 16)] = base
        rowB[pl.ds(j * 16, 16)] = base + HROWS
    @pl.when(s < 10)
    def _():
        pltpu.sync_copy(hist.at[pl.ds(0, 16)], accum.at[pl.ds(s * 16, 16)])
    plsc.subcore_barrier()

    def histogram(idx_hbm_ref, row_ref):
        pltpu.async_copy(idx_hbm_ref, idxbuf, sem).wait()

        def step(g, _):
            v = idxbuf[pl.ds(g * 16, 16)]
            hi = lax.shift_right_logical(v, 7)
            lo = lax.bitwise_and(v, 127)
            cnt, last = plsc.scan_count(v)
            val = cnt.astype(jnp.float32)
            plsc.addupdate_scatter(hist, [hi, lo], val, mask=last)
            return _
        lax.fori_loop(0, EPT // 16, step, None)
        pltpu.sync_copy(hist, accum.at[row_ref], add=True)

    histogram(src_ref.at[pl.ds(wid * EPT, EPT)], rowA)
    zero_hist()
    histogram(dst_ref.at[pl.ds(wid * EPT, EPT)], rowB)
    plsc.subcore_barrier()

    @pl.when(s < 10)
    def _():
        pltpu.sync_copy(accum.at[pl.ds(s * 16, 16)],
                        deg_ref.at[c, pl.ds(s * 16, 16)])


@functools.lru_cache(maxsize=None)
def _deg_kernel():
  return pl.kernel(
    _deg_body,
    out_type=jax.ShapeDtypeStruct((NC, 2 * HROWS, D), jnp.float32),
    mesh=_sc_mesh(),
    scratch_types=[
        pltpu.VMEM((HROWS, D), jnp.float32),
        pltpu.VMEM((EPT,), jnp.int32),
        pltpu.VMEM((HROWS,), jnp.int32),
        pltpu.VMEM((HROWS,), jnp.int32),
        pltpu.VMEM_SHARED((2 * HROWS, D), jnp.float32),
        pltpu.SemaphoreType.DMA,
    ],
    compiler_params=pltpu.CompilerParams(needs_layout_passes=False),
  )


def _b1_body(x_ref, ws_ref, bs_ref, wd_ref, bd_ref, deg_ref,
             e1_ref, e2_ref, fs_ref, nr_ref):
    x = x_ref[...]
    e1_ref[...] = jnp.dot(x, ws_ref[...], preferred_element_type=jnp.float32,
                          precision=lax.Precision.HIGHEST) + bs_ref[...][None, :]
    e2_ref[...] = jnp.dot(x, wd_ref[...], preferred_element_type=jnp.float32,
                          precision=lax.Precision.HIGHEST) + bd_ref[...][None, :]
    deg = deg_ref[...]
    out_deg = deg[0, 0] + deg[1, 0]
    in_deg = deg[0, 1] + deg[1, 1]
    norm_l = lax.rsqrt(jnp.maximum(out_deg, 1.0))
    nr_ref[...] = lax.rsqrt(jnp.maximum(in_deg, 1.0))
    fs_ref[...] = x * norm_l


def _run_b1(x_pad, W_src, b_src, W_dst, b_dst, deg4):
    f32 = jnp.float32
    return pl.pallas_call(
        _b1_body,
        grid=(GRID_N,),
        in_specs=[
            pl.BlockSpec((BR, D), lambda i: (i, 0)),
            pl.BlockSpec((D, D), lambda i: (0, 0)),
            pl.BlockSpec((D,), lambda i: (0,)),
            pl.BlockSpec((D, D), lambda i: (0, 0)),
            pl.BlockSpec((D,), lambda i: (0,)),
            pl.BlockSpec((NC, 2, BR, 1), lambda i: (0, 0, i, 0)),
        ],
        out_specs=[
            pl.BlockSpec((BR, D), lambda i: (i, 0)),
            pl.BlockSpec((BR, D), lambda i: (i, 0)),
            pl.BlockSpec((BR, D), lambda i: (i, 0)),
            pl.BlockSpec((BR, 1), lambda i: (i, 0)),
        ],
        out_shape=[
            jax.ShapeDtypeStruct((NPAD, D), f32),
            jax.ShapeDtypeStruct((NPAD, D), f32),
            jax.ShapeDtypeStruct((NPAD, D), f32),
            jax.ShapeDtypeStruct((NPAD, 1), f32),
        ],
    )(x_pad, W_src, b_src, W_dst, b_dst, deg4)


def _b2_body(ef_ref, we_ref, be_ref, et_ref):
    et_ref[...] = jnp.dot(ef_ref[...], we_ref[...],
                          preferred_element_type=jnp.float32,
                          precision=lax.Precision.HIGHEST) + be_ref[...][None, :]


def _run_b2(edge_feats, W_edge, b_edge):
    BE = 8000
    return pl.pallas_call(
        _b2_body,
        grid=(E // BE,),
        in_specs=[
            pl.BlockSpec((BE, D), lambda i: (i, 0)),
            pl.BlockSpec((D, D), lambda i: (0, 0)),
            pl.BlockSpec((D,), lambda i: (0,)),
        ],
        out_specs=pl.BlockSpec((BE, D), lambda i: (i, 0)),
        out_shape=jax.ShapeDtypeStruct((E, D), jnp.float32),
    )(edge_feats, W_edge, b_edge)


def _edge_body(src_ref, dst_ref, e1_ref, e2_ref, fs_ref, et_ref,
               m_ref, part_ref,
               bufA, bB, bET, bufF, sidx, didx, accum, sem):
    bufM = bufF
    c = lax.axis_index("c")
    s = lax.axis_index("s")
    wid = s * NC + c

    def zrow(i, _):
        for j in range(8):
            bufM[i, pl.ds(j * 16, 16)] = jnp.zeros((16,), jnp.float32)
        return _
    lax.fori_loop(0, CH, zrow, None)
    r0 = s * ROWS_PER_TILE
    nfull = ROWS_PER_TILE // CH
    remr = ROWS_PER_TILE % CH
    for r in range(nfull):
        pltpu.sync_copy(bufM, accum.at[pl.ds(r0 + r * CH, CH)])
    if remr:
        pltpu.sync_copy(bufM.at[pl.ds(0, remr)],
                        accum.at[pl.ds(r0 + nfull * CH, remr)])
    plsc.subcore_barrier()

    nchunks_w = CHUNKS_BASE + (wid < CHUNKS_REM).astype(jnp.int32)

    def chunk(j, _):
        eoff = (wid + NW * j) * CH
        pltpu.sync_copy(src_ref.at[pl.ds(eoff, CH)], sidx)
        pltpu.sync_copy(dst_ref.at[pl.ds(eoff, CH)], didx)
        d1 = pltpu.async_copy(e1_ref.at[sidx], bufA, sem)
        d2 = pltpu.async_copy(e2_ref.at[didx], bB, sem)
        d3 = pltpu.async_copy(fs_ref.at[sidx], bufF, sem)
        d4 = pltpu.async_copy(et_ref.at[pl.ds(eoff, CH)], bET, sem)
        d1.wait(); d2.wait(); d3.wait(); d4.wait()

        def row(i, _):
            for j8 in range(8):
                sl = pl.ds(j8 * 16, 16)
                m = bufA[i, sl] + bB[i, sl] + bET[i, sl]
                sig = 1.0 / (1.0 + jnp.exp(-m))
                bET[i, sl] = m
                bufM[i, sl] = bufF[i, sl] * sig
            return _
        lax.fori_loop(0, CH, row, None)

        pltpu.sync_copy(bET, m_ref.at[pl.ds(eoff, CH)])
        pltpu.sync_copy(bufM, accum.at[didx], add=True)
        return _
    lax.fori_loop(0, nchunks_w, chunk, None)

    plsc.subcore_barrier()
    pltpu.sync_copy(accum.at[pl.ds(r0, ROWS_PER_TILE)],
                    part_ref.at[c, pl.ds(r0, ROWS_PER_TILE)])


@functools.lru_cache(maxsize=None)
def _edge_kernel():
  f32 = jnp.float32
  return pl.kernel(
    _edge_body,
    out_type=(
        jax.ShapeDtypeStruct((E, D), f32),
        jax.ShapeDtypeStruct((NC, NPAD, D), f32),
    ),
    mesh=_sc_mesh(),
    scratch_types=[
        pltpu.VMEM((CH, D), f32),
        pltpu.VMEM((CH, D), f32),
        pltpu.VMEM((CH, D), f32),
        pltpu.VMEM((CH, D), f32),
        pltpu.VMEM((CH,), jnp.int32),
        pltpu.VMEM((CH,), jnp.int32),
        pltpu.VMEM_SHARED((NPAD, D), f32),
        pltpu.SemaphoreType.DMA,
    ],
  )


def _d_body(part_ref, w_ref, b_ref, nr_ref, x_ref, out_ref):
    p = part_ref[...]
    sacc = p[0] + p[1]
    r = jnp.dot(sacc, w_ref[...], preferred_element_type=jnp.float32,
                precision=lax.Precision.HIGHEST)
    out_ref[...] = r * nr_ref[...] + b_ref[...][None, :] + x_ref[...]


def _run_d(partials, weight, bias, nr, x_pad):
    return pl.pallas_call(
        _d_body,
        grid=(GRID_N,),
        in_specs=[
            pl.BlockSpec((NC, BR, D), lambda i: (0, i, 0)),
            pl.BlockSpec((D, D), lambda i: (0, 0)),
            pl.BlockSpec((D,), lambda i: (0,)),
            pl.BlockSpec((BR, 1), lambda i: (i, 0)),
            pl.BlockSpec((BR, D), lambda i: (i, 0)),
        ],
        out_specs=pl.BlockSpec((BR, D), lambda i: (i, 0)),
        out_shape=jax.ShapeDtypeStruct((NPAD, D), jnp.float32),
    )(partials, weight, bias, nr, x_pad)


@jax.jit
def kernel(node_feats, edge_index, edge_feats, weight, bias,
           W_src, b_src, W_dst, b_dst, W_edge, b_edge):
    src = edge_index[0].astype(jnp.int32)
    dst = edge_index[1].astype(jnp.int32)
    x_pad = jnp.pad(node_feats, ((0, NPAD - N), (0, 0)))

    deg = _deg_kernel()(src, dst)
    deg4 = deg.reshape(NC, 2, NPAD, 1)
    e1, e2, fs, nr = _run_b1(x_pad, W_src, b_src, W_dst, b_dst, deg4)
    et = _run_b2(edge_feats, W_edge, b_edge)
    m, partials = _edge_kernel()(src, dst, e1, e2, fs, et)
    rst_pad = _run_d(partials, weight, bias, nr, x_pad)
    return (rst_pad[:N], m)


# speedup vs baseline: 2.9688x; 1.1518x over previous
import functools
import jax
import jax.numpy as jnp
from jax import lax
from jax.experimental import pallas as pl
from jax.experimental.pallas import tpu as pltpu
from jax.experimental.pallas import tpu_sc as plsc

N = 10000
E = 320000
D = 128
NPAD = 10240
HROWS = 80
BR = 2048
GRID_N = NPAD // BR
NC = 2
NS = 16
NW = NC * NS
CH = 64
NCHUNKS = E // CH
CPW = NCHUNKS // NW
NEXTRA = NCHUNKS - CPW * NW
EXOFF = NW * CPW * CH
IDXB = 4
NBATCH = CPW // IDXB
ACC_ROWS = 10112
APT = ACC_ROWS // NS
EPT = E // NW
ROWS_PER_TILE = NPAD // NS

@functools.lru_cache(maxsize=None)
def _sc_mesh():
    return plsc.VectorSubcoreMesh(core_axis_name="c", subcore_axis_name="s",
                                  num_cores=NC, num_subcores=NS)


def _deg_body(src_ref, dst_ref, deg_ref, hist, idxbuf, rowA, rowB, accum, sem):
    c = lax.axis_index("c")
    s = lax.axis_index("s")
    wid = s * NC + c

    def zero_hist():
        def zrow(i, _):
            for j in range(8):
                hist[i, pl.ds(j * 16, 16)] = jnp.zeros((16,), jnp.float32)
            return _
        lax.fori_loop(0, HROWS, zrow, None)

    zero_hist()
    for j in range(5):
        base = lax.iota(jnp.int32, 16) + j * 16
        rowA[pl.ds(j * 16, 16)] = base
        rowB[pl.ds(j * 16, 16)] = base + HROWS
    @pl.when(s < 10)
    def _():
        pltpu.sync_copy(hist.at[pl.ds(0, 16)], accum.at[pl.ds(s * 16, 16)])
    plsc.subcore_barrier()

    def histogram(idx_hbm_ref, row_ref):
        pltpu.async_copy(idx_hbm_ref, idxbuf, sem).wait()

        def step(g, _):
            v = idxbuf[pl.ds(g * 16, 16)]
            hi = lax.shift_right_logical(v, 7)
            lo = lax.bitwise_and(v, 127)
            cnt, last = plsc.scan_count(v)
            val = cnt.astype(jnp.float32)
            plsc.addupdate_scatter(hist, [hi, lo], val, mask=last)
            return _
        lax.fori_loop(0, EPT // 16, step, None)
        pltpu.sync_copy(hist, accum.at[row_ref], add=True)

    histogram(src_ref.at[pl.ds(wid * EPT, EPT)], rowA)
    zero_hist()
    histogram(dst_ref.at[pl.ds(wid * EPT, EPT)], rowB)
    plsc.subcore_barrier()

    @pl.when(s < 10)
    def _():
        pltpu.sync_copy(accum.at[pl.ds(s * 16, 16)],
                        deg_ref.at[c, pl.ds(s * 16, 16)])


@functools.lru_cache(maxsize=None)
def _deg_kernel():
  return pl.kernel(
    _deg_body,
    out_type=jax.ShapeDtypeStruct((NC, 2 * HROWS, D), jnp.float32),
    mesh=_sc_mesh(),
    scratch_types=[
        pltpu.VMEM((HROWS, D), jnp.float32),
        pltpu.VMEM((EPT,), jnp.int32),
        pltpu.VMEM((HROWS,), jnp.int32),
        pltpu.VMEM((HROWS,), jnp.int32),
        pltpu.VMEM_SHARED((2 * HROWS, D), jnp.float32),
        pltpu.SemaphoreType.DMA,
    ],
    compiler_params=pltpu.CompilerParams(needs_layout_passes=False),
  )


def _b1_body(x_ref, ws_ref, bs_ref, wd_ref, bd_ref, deg_ref,
             e1_ref, e2_ref, fs_ref, nr_ref):
    x = x_ref[...]
    e1_ref[...] = jnp.dot(x, ws_ref[...], preferred_element_type=jnp.float32,
                          precision=lax.Precision.HIGHEST) + bs_ref[...][None, :]
    e2_ref[...] = jnp.dot(x, wd_ref[...], preferred_element_type=jnp.float32,
                          precision=lax.Precision.HIGHEST) + bd_ref[...][None, :]
    deg = deg_ref[...]
    out_deg = deg[0, 0] + deg[1, 0]
    in_deg = deg[0, 1] + deg[1, 1]
    norm_l = lax.rsqrt(jnp.maximum(out_deg, 1.0))
    nr_ref[...] = lax.rsqrt(jnp.maximum(in_deg, 1.0))
    fs_ref[...] = x * norm_l


def _run_b1(x_pad, W_src, b_src, W_dst, b_dst, deg4):
    f32 = jnp.float32
    return pl.pallas_call(
        _b1_body,
        grid=(GRID_N,),
        in_specs=[
            pl.BlockSpec((BR, D), lambda i: (i, 0)),
            pl.BlockSpec((D, D), lambda i: (0, 0)),
            pl.BlockSpec((D,), lambda i: (0,)),
            pl.BlockSpec((D, D), lambda i: (0, 0)),
            pl.BlockSpec((D,), lambda i: (0,)),
            pl.BlockSpec((NC, 2, BR, 1), lambda i: (0, 0, i, 0)),
        ],
        out_specs=[
            pl.BlockSpec((BR, D), lambda i: (i, 0)),
            pl.BlockSpec((BR, D), lambda i: (i, 0)),
            pl.BlockSpec((BR, D), lambda i: (i, 0)),
            pl.BlockSpec((BR, 1), lambda i: (i, 0)),
        ],
        out_shape=[
            jax.ShapeDtypeStruct((NPAD, D), f32),
            jax.ShapeDtypeStruct((NPAD, D), f32),
            jax.ShapeDtypeStruct((NPAD, D), f32),
            jax.ShapeDtypeStruct((NPAD, 1), f32),
        ],
    )(x_pad, W_src, b_src, W_dst, b_dst, deg4)


def _b2_body(ef_ref, we_ref, be_ref, et_ref):
    et_ref[...] = jnp.dot(ef_ref[...], we_ref[...],
                          preferred_element_type=jnp.float32,
                          precision=lax.Precision.HIGHEST) + be_ref[...][None, :]


def _run_b2(edge_feats, W_edge, b_edge):
    BE = 8000
    return pl.pallas_call(
        _b2_body,
        grid=(E // BE,),
        in_specs=[
            pl.BlockSpec((BE, D), lambda i: (i, 0)),
            pl.BlockSpec((D, D), lambda i: (0, 0)),
            pl.BlockSpec((D,), lambda i: (0,)),
        ],
        out_specs=pl.BlockSpec((BE, D), lambda i: (i, 0)),
        out_shape=jax.ShapeDtypeStruct((E, D), jnp.float32),
    )(edge_feats, W_edge, b_edge)


def _edge_body(src_ref, dst_ref, e1_ref, e2_ref, fs_ref, et_ref,
               m_ref, part_ref,
               bufA, bB, bET0, bET1, bufF0, bufF1,
               sbig, dbig, dsm0, dsm1, accum, semg, semst0, semst1):
    bETs = (bET0, bET1)
    bufFs = (bufF0, bufF1)
    dsms = (dsm0, dsm1)
    semsts = (semst0, semst1)
    c = lax.axis_index("c")
    s = lax.axis_index("s")
    wid = s * NC + c

    def zrow(i, _):
        for j in range(8):
            bufF0[i, pl.ds(j * 16, 16)] = jnp.zeros((16,), jnp.float32)
        return _
    lax.fori_loop(0, CH, zrow, None)
    r0 = s * APT
    nfull = APT // CH
    remr = APT % CH
    for r in range(nfull):
        pltpu.sync_copy(bufF0, accum.at[pl.ds(r0 + r * CH, CH)])
    if remr:
        pltpu.sync_copy(bufF0.at[pl.ds(0, remr)],
                        accum.at[pl.ds(r0 + nfull * CH, remr)])
    plsc.subcore_barrier()

    wbase = wid * CPW * CH

    def wait_stores(P):
        pltpu.make_async_copy(bETs[P], m_ref.at[pl.ds(0, CH)],
                              semsts[P]).wait()

    def do_chunk(eoff, koff, P):
        sidx = sbig.at[pl.ds(koff, CH)]
        didx = dbig.at[pl.ds(koff, CH)]
        d1 = pltpu.async_copy(e1_ref.at[sidx], bufA, semg)
        d2 = pltpu.async_copy(e2_ref.at[didx], bB, semg)
        d3 = pltpu.async_copy(fs_ref.at[sidx], bufFs[P], semg)
        d4 = pltpu.async_copy(et_ref.at[pl.ds(eoff, CH)], bETs[P], semg)
        for g in range(CH // 16):
            dsms[P][pl.ds(g * 16, 16)] = dbig[pl.ds(koff + g * 16, 16)]
        d1.wait(); d2.wait(); d3.wait(); d4.wait()

        bET = bETs[P]
        bufF = bufFs[P]

        def row(i, _):
            for j8 in range(8):
                sl = pl.ds(j8 * 16, 16)
                m = bufA[i, sl] + bB[i, sl] + bET[i, sl]
                sig = 1.0 / (1.0 + jnp.exp(-m))
                bET[i, sl] = m
                bufF[i, sl] = bufF[i, sl] * sig
            return _
        lax.fori_loop(0, CH, row, None)

        pltpu.async_copy(bET, m_ref.at[pl.ds(eoff, CH)], semsts[P])
        pltpu.sync_copy(bufF, accum.at[dsms[P]], add=True)

    def batch(b, _):
        boff = wbase + b * IDXB * CH
        pltpu.sync_copy(src_ref.at[pl.ds(boff, IDXB * CH)], sbig)
        pltpu.sync_copy(dst_ref.at[pl.ds(boff, IDXB * CH)], dbig)
        for k in range(IDXB):
            P = k % 2
            if k >= 2:
                wait_stores(P)
            else:
                @pl.when(b > 0)
                def _():
                    wait_stores(P)
            do_chunk(boff + k * CH, k * CH, P)
        return _
    lax.fori_loop(0, NBATCH, batch, None)

    @pl.when(wid < NEXTRA)
    def _():
        eoff = EXOFF + wid * CH
        wait_stores(0)
        pltpu.sync_copy(src_ref.at[pl.ds(eoff, CH)], sbig.at[pl.ds(0, CH)])
        pltpu.sync_copy(dst_ref.at[pl.ds(eoff, CH)], dbig.at[pl.ds(0, CH)])
        do_chunk(eoff, 0, 0)

    wait_stores(0)
    wait_stores(1)
    plsc.subcore_barrier()
    pltpu.sync_copy(accum.at[pl.ds(r0, APT)],
                    part_ref.at[c, pl.ds(r0, APT)])


@functools.lru_cache(maxsize=None)
def _edge_kernel():
  f32 = jnp.float32
  return pl.kernel(
    _edge_body,
    out_type=(
        jax.ShapeDtypeStruct((E, D), f32),
        jax.ShapeDtypeStruct((NC, NPAD, D), f32),
    ),
    mesh=_sc_mesh(),
    scratch_types=[
        pltpu.VMEM((CH, D), f32),
        pltpu.VMEM((CH, D), f32),
        pltpu.VMEM((CH, D), f32),
        pltpu.VMEM((CH, D), f32),
        pltpu.VMEM((CH, D), f32),
        pltpu.VMEM((CH, D), f32),
        pltpu.VMEM((IDXB * CH,), jnp.int32),
        pltpu.VMEM((IDXB * CH,), jnp.int32),
        pltpu.VMEM((CH,), jnp.int32),
        pltpu.VMEM((CH,), jnp.int32),
        pltpu.VMEM_SHARED((ACC_ROWS, D), f32),
        pltpu.SemaphoreType.DMA,
        pltpu.SemaphoreType.DMA,
        pltpu.SemaphoreType.DMA,
    ],
  )


def _d_body(part_ref, w_ref, b_ref, nr_ref, x_ref, out_ref):
    p = part_ref[...]
    sacc = p[0] + p[1]
    r = jnp.dot(sacc, w_ref[...], preferred_element_type=jnp.float32,
                precision=lax.Precision.HIGHEST)
    out_ref[...] = r * nr_ref[...] + b_ref[...][None, :] + x_ref[...]


def _run_d(partials, weight, bias, nr, x_pad):
    return pl.pallas_call(
        _d_body,
        grid=(GRID_N,),
        in_specs=[
            pl.BlockSpec((NC, BR, D), lambda i: (0, i, 0)),
            pl.BlockSpec((D, D), lambda i: (0, 0)),
            pl.BlockSpec((D,), lambda i: (0,)),
            pl.BlockSpec((BR, 1), lambda i: (i, 0)),
            pl.BlockSpec((BR, D), lambda i: (i, 0)),
        ],
        out_specs=pl.BlockSpec((BR, D), lambda i: (i, 0)),
        out_shape=jax.ShapeDtypeStruct((NPAD, D), jnp.float32),
    )(partials, weight, bias, nr, x_pad)


@jax.jit
def kernel(node_feats, edge_index, edge_feats, weight, bias,
           W_src, b_src, W_dst, b_dst, W_edge, b_edge):
    src = edge_index[0].astype(jnp.int32)
    dst = edge_index[1].astype(jnp.int32)
    x_pad = jnp.pad(node_feats, ((0, NPAD - N), (0, 0)))

    deg = _deg_kernel()(src, dst)
    deg4 = deg.reshape(NC, 2, NPAD, 1)
    e1, e2, fs, nr = _run_b1(x_pad, W_src, b_src, W_dst, b_dst, deg4)
    et = _run_b2(edge_feats, W_edge, b_edge)
    m, partials = _edge_kernel()(src, dst, e1, e2, fs, et)
    rst_pad = _run_d(partials, weight, bias, nr, x_pad)
    return (rst_pad[:N], m)


# speedup vs baseline: 3.0183x; 1.0167x over previous
import functools
import jax
import jax.numpy as jnp
from jax import lax
from jax.experimental import pallas as pl
from jax.experimental.pallas import tpu as pltpu
from jax.experimental.pallas import tpu_sc as plsc

N = 10000
E = 320000
D = 128
NPAD = 10240
HROWS = 80
BR = 2048
GRID_N = NPAD // BR
NC = 2
NS = 16
NW = NC * NS
CH = 64
NCHUNKS = E // CH
CPW = NCHUNKS // NW
NEXTRA = NCHUNKS - CPW * NW
EXOFF = NW * CPW * CH
IDXB = 6
NBATCH = CPW // IDXB
ACC_ROWS = 10112
APT = ACC_ROWS // NS
EPT = E // NW
ROWS_PER_TILE = NPAD // NS

@functools.lru_cache(maxsize=None)
def _sc_mesh():
    return plsc.VectorSubcoreMesh(core_axis_name="c", subcore_axis_name="s",
                                  num_cores=NC, num_subcores=NS)


def _deg_body(src_ref, dst_ref, deg_ref, hist, idxbuf, rowA, rowB, accum, sem):
    c = lax.axis_index("c")
    s = lax.axis_index("s")
    wid = s * NC + c

    def zero_hist():
        def zrow(i, _):
            for j in range(8):
                hist[i, pl.ds(j * 16, 16)] = jnp.zeros((16,), jnp.float32)
            return _
        lax.fori_loop(0, HROWS, zrow, None)

    zero_hist()
    for j in range(5):
        base = lax.iota(jnp.int32, 16) + j * 16
        rowA[pl.ds(j * 16, 16)] = base
        rowB[pl.ds(j * 16, 16)] = base + HROWS
    @pl.when(s < 10)
    def _():
        pltpu.sync_copy(hist.at[pl.ds(0, 16)], accum.at[pl.ds(s * 16, 16)])
    plsc.subcore_barrier()

    def histogram(idx_hbm_ref, row_ref):
        pltpu.async_copy(idx_hbm_ref, idxbuf, sem).wait()

        def step(g, _):
            v = idxbuf[pl.ds(g * 16, 16)]
            hi = lax.shift_right_logical(v, 7)
            lo = lax.bitwise_and(v, 127)
            cnt, last = plsc.scan_count(v)
            val = cnt.astype(jnp.float32)
            plsc.addupdate_scatter(hist, [hi, lo], val, mask=last)
            return _
        lax.fori_loop(0, EPT // 16, step, None)
        pltpu.sync_copy(hist, accum.at[row_ref], add=True)

    histogram(src_ref.at[pl.ds(wid * EPT, EPT)], rowA)
    zero_hist()
    histogram(dst_ref.at[pl.ds(wid * EPT, EPT)], rowB)
    plsc.subcore_barrier()

    @pl.when(s < 10)
    def _():
        pltpu.sync_copy(accum.at[pl.ds(s * 16, 16)],
                        deg_ref.at[c, pl.ds(s * 16, 16)])


@functools.lru_cache(maxsize=None)
def _deg_kernel():
  return pl.kernel(
    _deg_body,
    out_type=jax.ShapeDtypeStruct((NC, 2 * HROWS, D), jnp.float32),
    mesh=_sc_mesh(),
    scratch_types=[
        pltpu.VMEM((HROWS, D), jnp.float32),
        pltpu.VMEM((EPT,), jnp.int32),
        pltpu.VMEM((HROWS,), jnp.int32),
        pltpu.VMEM((HROWS,), jnp.int32),
        pltpu.VMEM_SHARED((2 * HROWS, D), jnp.float32),
        pltpu.SemaphoreType.DMA,
    ],
    compiler_params=pltpu.CompilerParams(needs_layout_passes=False),
  )


def _b1_body(x_ref, ws_ref, bs_ref, wd_ref, bd_ref, deg_ref,
             e1_ref, e2_ref, fs_ref, nr_ref):
    x = x_ref[...]
    e1_ref[...] = jnp.dot(x, ws_ref[...], preferred_element_type=jnp.float32,
                          precision=lax.Precision.HIGHEST) + bs_ref[...][None, :]
    e2_ref[...] = jnp.dot(x, wd_ref[...], preferred_element_type=jnp.float32,
                          precision=lax.Precision.HIGHEST) + bd_ref[...][None, :]
    deg = deg_ref[...]
    out_deg = deg[0, 0] + deg[1, 0]
    in_deg = deg[0, 1] + deg[1, 1]
    norm_l = lax.rsqrt(jnp.maximum(out_deg, 1.0))
    nr_ref[...] = lax.rsqrt(jnp.maximum(in_deg, 1.0))
    fs_ref[...] = x * norm_l


def _run_b1(x_pad, W_src, b_src, W_dst, b_dst, deg4):
    f32 = jnp.float32
    return pl.pallas_call(
        _b1_body,
        grid=(GRID_N,),
        in_specs=[
            pl.BlockSpec((BR, D), lambda i: (i, 0)),
            pl.BlockSpec((D, D), lambda i: (0, 0)),
            pl.BlockSpec((D,), lambda i: (0,)),
            pl.BlockSpec((D, D), lambda i: (0, 0)),
            pl.BlockSpec((D,), lambda i: (0,)),
            pl.BlockSpec((NC, 2, BR, 1), lambda i: (0, 0, i, 0)),
        ],
        out_specs=[
            pl.BlockSpec((BR, D), lambda i: (i, 0)),
            pl.BlockSpec((BR, D), lambda i: (i, 0)),
            pl.BlockSpec((BR, D), lambda i: (i, 0)),
            pl.BlockSpec((BR, 1), lambda i: (i, 0)),
        ],
        out_shape=[
            jax.ShapeDtypeStruct((NPAD, D), f32),
            jax.ShapeDtypeStruct((NPAD, D), f32),
            jax.ShapeDtypeStruct((NPAD, D), f32),
            jax.ShapeDtypeStruct((NPAD, 1), f32),
        ],
    )(x_pad, W_src, b_src, W_dst, b_dst, deg4)


def _b2_body(ef_ref, we_ref, be_ref, et_ref):
    et_ref[...] = jnp.dot(ef_ref[...], we_ref[...],
                          preferred_element_type=jnp.float32,
                          precision=lax.Precision.HIGHEST) + be_ref[...][None, :]


def _run_b2(edge_feats, W_edge, b_edge):
    BE = 8000
    return pl.pallas_call(
        _b2_body,
        grid=(E // BE,),
        in_specs=[
            pl.BlockSpec((BE, D), lambda i: (i, 0)),
            pl.BlockSpec((D, D), lambda i: (0, 0)),
            pl.BlockSpec((D,), lambda i: (0,)),
        ],
        out_specs=pl.BlockSpec((BE, D), lambda i: (i, 0)),
        out_shape=jax.ShapeDtypeStruct((E, D), jnp.float32),
    )(edge_feats, W_edge, b_edge)


def _edge_body(src_ref, dst_ref, e1_ref, e2_ref, fs_ref, et_ref,
               m_ref, part_ref,
               bufA, bB, bET0, bET1, bufF0, bufF1,
               sbig, dbig, dsm0, dsm1, accum, semg, semst0, semst1):
    bETs = (bET0, bET1)
    bufFs = (bufF0, bufF1)
    dsms = (dsm0, dsm1)
    semsts = (semst0, semst1)
    c = lax.axis_index("c")
    s = lax.axis_index("s")
    wid = s * NC + c

    def zrow(i, _):
        for j in range(8):
            bufF0[i, pl.ds(j * 16, 16)] = jnp.zeros((16,), jnp.float32)
        return _
    lax.fori_loop(0, CH, zrow, None)
    r0 = s * APT
    nfull = APT // CH
    remr = APT % CH
    for r in range(nfull):
        pltpu.sync_copy(bufF0, accum.at[pl.ds(r0 + r * CH, CH)])
    if remr:
        pltpu.sync_copy(bufF0.at[pl.ds(0, remr)],
                        accum.at[pl.ds(r0 + nfull * CH, remr)])
    plsc.subcore_barrier()

    wbase = wid * CPW * CH

    def wait_stores(P):
        pltpu.make_async_copy(bETs[P], m_ref.at[pl.ds(0, CH)],
                              semsts[P]).wait()

    def do_chunk(eoff, koff, P):
        sidx = sbig.at[pl.ds(koff, CH)]
        didx = dbig.at[pl.ds(koff, CH)]
        d1 = pltpu.async_copy(e1_ref.at[sidx], bufA, semg)
        d2 = pltpu.async_copy(e2_ref.at[didx], bB, semg)
        d3 = pltpu.async_copy(fs_ref.at[sidx], bufFs[P], semg)
        d4 = pltpu.async_copy(et_ref.at[pl.ds(eoff, CH)], bETs[P], semg)
        for g in range(CH // 16):
            dsms[P][pl.ds(g * 16, 16)] = dbig[pl.ds(koff + g * 16, 16)]
        d1.wait(); d2.wait(); d3.wait(); d4.wait()

        bET = bETs[P]
        bufF = bufFs[P]

        def row(i, _):
            for j8 in range(8):
                sl = pl.ds(j8 * 16, 16)
                m = bufA[i, sl] + bB[i, sl] + bET[i, sl]
                sig = 1.0 / (1.0 + jnp.exp(-m))
                bET[i, sl] = m
                bufF[i, sl] = bufF[i, sl] * sig
            return _
        lax.fori_loop(0, CH, row, None)

        pltpu.async_copy(bET, m_ref.at[pl.ds(eoff, CH)], semsts[P])
        pltpu.sync_copy(bufF, accum.at[dsms[P]], add=True)

    def batch(b, _):
        boff = wbase + b * IDXB * CH
        pltpu.sync_copy(src_ref.at[pl.ds(boff, IDXB * CH)], sbig)
        pltpu.sync_copy(dst_ref.at[pl.ds(boff, IDXB * CH)], dbig)
        for k in range(IDXB):
            P = k % 2
            if k >= 2:
                wait_stores(P)
            else:
                @pl.when(b > 0)
                def _():
                    wait_stores(P)
            do_chunk(boff + k * CH, k * CH, P)
        return _
    lax.fori_loop(0, NBATCH, batch, None)

    @pl.when(wid < NEXTRA)
    def _():
        eoff = EXOFF + wid * CH
        wait_stores(0)
        pltpu.sync_copy(src_ref.at[pl.ds(eoff, CH)], sbig.at[pl.ds(0, CH)])
        pltpu.sync_copy(dst_ref.at[pl.ds(eoff, CH)], dbig.at[pl.ds(0, CH)])
        do_chunk(eoff, 0, 0)

    wait_stores(0)
    wait_stores(1)
    plsc.subcore_barrier()
    pltpu.sync_copy(accum.at[pl.ds(r0, APT)],
                    part_ref.at[c, pl.ds(r0, APT)])


@functools.lru_cache(maxsize=None)
def _edge_kernel():
  f32 = jnp.float32
  return pl.kernel(
    _edge_body,
    out_type=(
        jax.ShapeDtypeStruct((E, D), f32),
        jax.ShapeDtypeStruct((NC, NPAD, D), f32),
    ),
    mesh=_sc_mesh(),
    scratch_types=[
        pltpu.VMEM((CH, D), f32),
        pltpu.VMEM((CH, D), f32),
        pltpu.VMEM((CH, D), f32),
        pltpu.VMEM((CH, D), f32),
        pltpu.VMEM((CH, D), f32),
        pltpu.VMEM((CH, D), f32),
        pltpu.VMEM((IDXB * CH,), jnp.int32),
        pltpu.VMEM((IDXB * CH,), jnp.int32),
        pltpu.VMEM((CH,), jnp.int32),
        pltpu.VMEM((CH,), jnp.int32),
        pltpu.VMEM_SHARED((ACC_ROWS, D), f32),
        pltpu.SemaphoreType.DMA,
        pltpu.SemaphoreType.DMA,
        pltpu.SemaphoreType.DMA,
    ],
  )


def _d_body(part_ref, w_ref, b_ref, nr_ref, x_ref, out_ref):
    p = part_ref[...]
    sacc = p[0] + p[1]
    r = jnp.dot(sacc, w_ref[...], preferred_element_type=jnp.float32,
                precision=lax.Precision.HIGHEST)
    out_ref[...] = r * nr_ref[...] + b_ref[...][None, :] + x_ref[...]


def _run_d(partials, weight, bias, nr, x_pad):
    return pl.pallas_call(
        _d_body,
        grid=(GRID_N,),
        in_specs=[
            pl.BlockSpec((NC, BR, D), lambda i: (0, i, 0)),
            pl.BlockSpec((D, D), lambda i: (0, 0)),
            pl.BlockSpec((D,), lambda i: (0,)),
            pl.BlockSpec((BR, 1), lambda i: (i, 0)),
            pl.BlockSpec((BR, D), lambda i: (i, 0)),
        ],
        out_specs=pl.BlockSpec((BR, D), lambda i: (i, 0)),
        out_shape=jax.ShapeDtypeStruct((NPAD, D), jnp.float32),
    )(partials, weight, bias, nr, x_pad)


@jax.jit
def kernel(node_feats, edge_index, edge_feats, weight, bias,
           W_src, b_src, W_dst, b_dst, W_edge, b_edge):
    src = edge_index[0].astype(jnp.int32)
    dst = edge_index[1].astype(jnp.int32)
    x_pad = jnp.pad(node_feats, ((0, NPAD - N), (0, 0)))

    deg = _deg_kernel()(src, dst)
    deg4 = deg.reshape(NC, 2, NPAD, 1)
    e1, e2, fs, nr = _run_b1(x_pad, W_src, b_src, W_dst, b_dst, deg4)
    et = _run_b2(edge_feats, W_edge, b_edge)
    m, partials = _edge_kernel()(src, dst, e1, e2, fs, et)
    rst_pad = _run_d(partials, weight, bias, nr, x_pad)
    return (rst_pad[:N], m)
